# baseline (speedup 1.0000x reference)
import jax
import jax.numpy as jnp
from jax import lax
from jax.experimental import pallas as pl
from jax.experimental.pallas import tpu as pltpu
from jax.experimental.pallas import tpu_sc as plsc

N_NODES = 100000
D = 128
N_POOL = 200000
NC, NS = 2, 16
NW = NC * NS

NCW = 4
XC = 400
XCPW = 62
XSPAN = XC * XCPW
XTAIL = (N_NODES - NCW * XSPAN) // XC

PC = 128
CPWC = 26
CPWP = 52
CSPAN = CPWC * PC
PSPAN = CPWP * PC
PBASE = NCW * CSPAN
PT = 80
NTAIL = (N_POOL - PBASE - (NW - NCW) * PSPAN) // PT


def _sc_body(x_hbm, i0_hbm, i1_hbm, out_hbm,
             i0v, i1v, a_v, b_v, xsh, gsem0, gsem1, ssem0, ssem1):
    w = lax.axis_index("s") * NC + lax.axis_index("c")
    gsem = [gsem0, gsem1]
    ssem = [ssem0, ssem1]
    av = [a_v.at[0], a_v.at[1]]
    bv = [b_v.at[0], b_v.at[1]]

    def run_pool(base, cpw):
        pltpu.sync_copy(i0_hbm.at[pl.ds(base, cpw * PC)],
                        i0v.at[pl.ds(0, cpw * PC)])
        pltpu.sync_copy(i1_hbm.at[pl.ds(base, cpw * PC)],
                        i1v.at[pl.ds(0, cpw * PC)])

        def gather_descs(k, p):
            off = k * PC
            return [
                pltpu.make_async_copy(x_hbm.at[i0v.at[pl.ds(off, PC)]],
                                      av[p], gsem[p]),
                pltpu.make_async_copy(x_hbm.at[i1v.at[pl.ds(off, PC)]],
                                      bv[p], gsem[p]),
            ]

        def fire(k, p):
            for d in gather_descs(k, p):
                d.start()

        def compute(p):
            def row(i, carry):
                for j in range(D // 16):
                    s = pl.ds(j * 16, 16)
                    a_v[p, i, s] = (a_v[p, i, s] + b_v[p, i, s]) * 0.5
                return carry

            lax.fori_loop(0, PC, row, 0)

        def consume(k, p):
            for d in gather_descs(k, p):
                d.wait()
            compute(p)
            pltpu.async_copy(av[p],
                             out_hbm.at[pl.ds(N_NODES + base + k * PC, PC), :],
                             ssem[p])

        def wait_store(k, p):
            pltpu.make_async_copy(av[p],
                                  out_hbm.at[pl.ds(N_NODES + base + k * PC, PC), :],
                                  ssem[p]).wait()

        fire(0, 0)

        def pipe(t, carry):
            k1 = 2 * t + 1

            @pl.when(t >= 1)
            def _():
                wait_store(k1 - 2, 1)

            fire(k1, 1)
            consume(k1 - 1, 0)

            k2 = 2 * t + 2
            wait_store(k2 - 2, 0)
            fire(k2, 0)
            consume(k2 - 1, 1)
            return carry

        lax.fori_loop(0, (cpw - 2) // 2, pipe, 0)
        wait_store(cpw - 3, 1)
        fire(cpw - 1, 1)
        consume(cpw - 2, 0)
        consume(cpw - 1, 1)
        wait_store(cpw - 2, 0)
        wait_store(cpw - 1, 1)

    @pl.when(w < NCW)
    def _():
        base = w * XSPAN
        sidx = lax.axis_index("s")
        xb = [xsh.at[sidx, 0], xsh.at[sidx, 1]]

        def fire(k, p):
            pltpu.async_copy(x_hbm.at[pl.ds(base + k * XC, XC), :], xb[p],
                             gsem[p])

        def consume(k, p):
            pltpu.make_async_copy(x_hbm.at[pl.ds(base + k * XC, XC), :],
                                  xb[p], gsem[p]).wait()
            pltpu.async_copy(xb[p], out_hbm.at[pl.ds(base + k * XC, XC), :],
                             ssem[p])

        def wait_store(k, p):
            pltpu.make_async_copy(xb[p],
                                  out_hbm.at[pl.ds(base + k * XC, XC), :],
                                  ssem[p]).wait()

        fire(0, 0)

        def pipe(t, carry):
            k1 = 2 * t + 1

            @pl.when(t >= 1)
            def _():
                wait_store(k1 - 2, 1)

            fire(k1, 1)
            consume(k1 - 1, 0)

            k2 = 2 * t + 2
            wait_store(k2 - 2, 0)
            fire(k2, 0)
            consume(k2 - 1, 1)
            return carry

        lax.fori_loop(0, (XCPW - 2) // 2, pipe, 0)
        wait_store(XCPW - 3, 1)
        fire(XCPW - 1, 1)
        consume(XCPW - 2, 0)
        consume(XCPW - 1, 1)
        wait_store(XCPW - 2, 0)
        wait_store(XCPW - 1, 1)

        @pl.when(w < XTAIL)
        def _():
            tb = NCW * XSPAN + w * XC
            pltpu.sync_copy(x_hbm.at[pl.ds(tb, XC), :], xb[0])
            pltpu.sync_copy(xb[0], out_hbm.at[pl.ds(tb, XC), :])

        run_pool(w * CSPAN, CPWC)

    @pl.when(w >= NCW)
    def _():
        run_pool(PBASE + (w - NCW) * PSPAN, CPWP)

        @pl.when(w - NCW < NTAIL)
        def _():
            wt = w - NCW
            tbase = PBASE + (NW - NCW) * PSPAN + wt * PT
            av0 = a_v.at[0, pl.ds(0, PT), :]
            bv0 = b_v.at[0, pl.ds(0, PT), :]
            pltpu.sync_copy(i0_hbm.at[pl.ds(tbase, PT)], i0v.at[pl.ds(0, PT)])
            pltpu.sync_copy(i1_hbm.at[pl.ds(tbase, PT)], i1v.at[pl.ds(0, PT)])
            pltpu.async_copy(x_hbm.at[i0v.at[pl.ds(0, PT)]], av0, gsem[0])
            pltpu.async_copy(x_hbm.at[i1v.at[pl.ds(0, PT)]], bv0, gsem[0])
            pltpu.make_async_copy(x_hbm.at[i0v.at[pl.ds(0, PT)]], av0,
                                  gsem[0]).wait()
            pltpu.make_async_copy(x_hbm.at[i1v.at[pl.ds(0, PT)]], bv0,
                                  gsem[0]).wait()

            def trow(i, carry):
                for j in range(D // 16):
                    s = pl.ds(j * 16, 16)
                    a_v[0, i, s] = (a_v[0, i, s] + b_v[0, i, s]) * 0.5
                return carry

            lax.fori_loop(0, PT, trow, 0)
            pltpu.sync_copy(av0, out_hbm.at[pl.ds(N_NODES + tbase, PT), :])


def kernel(X, pool_idx):
    idx0 = pool_idx[:, 0]
    idx1 = pool_idx[:, 1]
    mesh = plsc.VectorSubcoreMesh(core_axis_name="c", subcore_axis_name="s")
    f = pl.kernel(
        _sc_body,
        out_type=jax.ShapeDtypeStruct((N_NODES + N_POOL, D), jnp.float32),
        mesh=mesh,
        scratch_types=[
            pltpu.VMEM((CPWP * PC,), jnp.int32),
            pltpu.VMEM((CPWP * PC,), jnp.int32),
            pltpu.VMEM((2, PC, D), jnp.float32),
            pltpu.VMEM((2, PC, D), jnp.float32),
            pltpu.VMEM_SHARED((NCW // NC, 2, XC, D), jnp.float32),
            pltpu.SemaphoreType.DMA,
            pltpu.SemaphoreType.DMA,
            pltpu.SemaphoreType.DMA,
            pltpu.SemaphoreType.DMA,
        ],
    )
    return f(X, idx0, idx1)


# speedup vs baseline: 1.0694x; 1.0694x over previous
import jax
import jax.numpy as jnp
from jax import lax
from jax.experimental import pallas as pl
from jax.experimental.pallas import tpu as pltpu
from jax.experimental.pallas import tpu_sc as plsc

N_NODES = 100000
D = 128
N_POOL = 200000
NC, NS = 2, 16
NW = NC * NS

NCW = 8
XC = 400
XCPW = 31
XSPAN = XC * XCPW
XTAIL = (N_NODES - NCW * XSPAN) // XC

PC = 128
CPWC = 33
CPWP = 54
CSPAN = CPWC * PC
PSPAN = CPWP * PC
PBASE = NCW * CSPAN
PT = 80
NTAIL = (N_POOL - PBASE - (NW - NCW) * PSPAN) // PT


def _sc_body(x_hbm, i0_hbm, i1_hbm, out_hbm,
             i0v, i1v, a_v, b_v, xsh, gsem0, gsem1, ssem0, ssem1):
    w = lax.axis_index("s") * NC + lax.axis_index("c")
    gsem = [gsem0, gsem1]
    ssem = [ssem0, ssem1]
    av = [a_v.at[0], a_v.at[1]]
    bv = [b_v.at[0], b_v.at[1]]

    def run_pool(base, cpw):
        pltpu.sync_copy(i0_hbm.at[pl.ds(base, cpw * PC)],
                        i0v.at[pl.ds(0, cpw * PC)])
        pltpu.sync_copy(i1_hbm.at[pl.ds(base, cpw * PC)],
                        i1v.at[pl.ds(0, cpw * PC)])

        def gather_descs(k, p):
            off = k * PC
            return [
                pltpu.make_async_copy(x_hbm.at[i0v.at[pl.ds(off, PC)]],
                                      av[p], gsem[p]),
                pltpu.make_async_copy(x_hbm.at[i1v.at[pl.ds(off, PC)]],
                                      bv[p], gsem[p]),
            ]

        def fire(k, p):
            for d in gather_descs(k, p):
                d.start()

        def compute(p):
            def row(i, carry):
                for j in range(D // 16):
                    s = pl.ds(j * 16, 16)
                    a_v[p, i, s] = (a_v[p, i, s] + b_v[p, i, s]) * 0.5
                return carry

            lax.fori_loop(0, PC, row, 0)

        def consume(k, p):
            for d in gather_descs(k, p):
                d.wait()
            compute(p)
            pltpu.async_copy(av[p],
                             out_hbm.at[pl.ds(N_NODES + base + k * PC, PC), :],
                             ssem[p])

        def wait_store(k, p):
            pltpu.make_async_copy(av[p],
                                  out_hbm.at[pl.ds(N_NODES + base + k * PC, PC), :],
                                  ssem[p]).wait()

        fire(0, 0)

        def pipe(t, carry):
            k1 = 2 * t + 1

            @pl.when(t >= 1)
            def _():
                wait_store(k1 - 2, 1)

            fire(k1, 1)
            consume(k1 - 1, 0)

            k2 = 2 * t + 2
            wait_store(k2 - 2, 0)
            fire(k2, 0)
            consume(k2 - 1, 1)
            return carry

        if cpw % 2 == 0:
            lax.fori_loop(0, (cpw - 2) // 2, pipe, 0)
            wait_store(cpw - 3, 1)
            fire(cpw - 1, 1)
            consume(cpw - 2, 0)
            consume(cpw - 1, 1)
            wait_store(cpw - 2, 0)
            wait_store(cpw - 1, 1)
        else:
            lax.fori_loop(0, (cpw - 1) // 2, pipe, 0)
            consume(cpw - 1, 0)
            wait_store(cpw - 2, 1)
            wait_store(cpw - 1, 0)

    @pl.when(w < NCW)
    def _():
        base = w * XSPAN
        sidx = lax.axis_index("s")
        xb = [xsh.at[sidx, 0], xsh.at[sidx, 1]]

        def fire(k, p):
            pltpu.async_copy(x_hbm.at[pl.ds(base + k * XC, XC), :], xb[p],
                             gsem[p])

        def consume(k, p):
            pltpu.make_async_copy(x_hbm.at[pl.ds(base + k * XC, XC), :],
                                  xb[p], gsem[p]).wait()
            pltpu.async_copy(xb[p], out_hbm.at[pl.ds(base + k * XC, XC), :],
                             ssem[p])

        def wait_store(k, p):
            pltpu.make_async_copy(xb[p],
                                  out_hbm.at[pl.ds(base + k * XC, XC), :],
                                  ssem[p]).wait()

        fire(0, 0)

        def pipe(t, carry):
            k1 = 2 * t + 1

            @pl.when(t >= 1)
            def _():
                wait_store(k1 - 2, 1)

            fire(k1, 1)
            consume(k1 - 1, 0)

            k2 = 2 * t + 2
            wait_store(k2 - 2, 0)
            fire(k2, 0)
            consume(k2 - 1, 1)
            return carry

        lax.fori_loop(0, (XCPW - 1) // 2, pipe, 0)
        consume(XCPW - 1, 0)
        wait_store(XCPW - 2, 1)
        wait_store(XCPW - 1, 0)

        @pl.when(w < XTAIL)
        def _():
            tb = NCW * XSPAN + w * XC
            pltpu.sync_copy(x_hbm.at[pl.ds(tb, XC), :], xb[0])
            pltpu.sync_copy(xb[0], out_hbm.at[pl.ds(tb, XC), :])

        run_pool(w * CSPAN, CPWC)

    @pl.when(w >= NCW)
    def _():
        run_pool(PBASE + (w - NCW) * PSPAN, CPWP)

        @pl.when(w - NCW < NTAIL)
        def _():
            wt = w - NCW
            tbase = PBASE + (NW - NCW) * PSPAN + wt * PT
            av0 = a_v.at[0, pl.ds(0, PT), :]
            bv0 = b_v.at[0, pl.ds(0, PT), :]
            pltpu.sync_copy(i0_hbm.at[pl.ds(tbase, PT)], i0v.at[pl.ds(0, PT)])
            pltpu.sync_copy(i1_hbm.at[pl.ds(tbase, PT)], i1v.at[pl.ds(0, PT)])
            pltpu.async_copy(x_hbm.at[i0v.at[pl.ds(0, PT)]], av0, gsem[0])
            pltpu.async_copy(x_hbm.at[i1v.at[pl.ds(0, PT)]], bv0, gsem[0])
            pltpu.make_async_copy(x_hbm.at[i0v.at[pl.ds(0, PT)]], av0,
                                  gsem[0]).wait()
            pltpu.make_async_copy(x_hbm.at[i1v.at[pl.ds(0, PT)]], bv0,
                                  gsem[0]).wait()

            def trow(i, carry):
                for j in range(D // 16):
                    s = pl.ds(j * 16, 16)
                    a_v[0, i, s] = (a_v[0, i, s] + b_v[0, i, s]) * 0.5
                return carry

            lax.fori_loop(0, PT, trow, 0)
            pltpu.sync_copy(av0, out_hbm.at[pl.ds(N_NODES + tbase, PT), :])


def kernel(X, pool_idx):
    idx0 = pool_idx[:, 0]
    idx1 = pool_idx[:, 1]
    mesh = plsc.VectorSubcoreMesh(core_axis_name="c", subcore_axis_name="s")
    f = pl.kernel(
        _sc_body,
        out_type=jax.ShapeDtypeStruct((N_NODES + N_POOL, D), jnp.float32),
        mesh=mesh,
        scratch_types=[
            pltpu.VMEM((CPWP * PC,), jnp.int32),
            pltpu.VMEM((CPWP * PC,), jnp.int32),
            pltpu.VMEM((2, PC, D), jnp.float32),
            pltpu.VMEM((2, PC, D), jnp.float32),
            pltpu.VMEM_SHARED((NCW // NC, 2, XC, D), jnp.float32),
            pltpu.SemaphoreType.DMA,
            pltpu.SemaphoreType.DMA,
            pltpu.SemaphoreType.DMA,
            pltpu.SemaphoreType.DMA,
        ],
    )
    return f(X, idx0, idx1)


# speedup vs baseline: 1.0950x; 1.0239x over previous
import jax
import jax.numpy as jnp
from jax import lax
from jax.experimental import pallas as pl
from jax.experimental.pallas import tpu as pltpu
from jax.experimental.pallas import tpu_sc as plsc

N_NODES = 100000
D = 128
N_POOL = 200000
NC, NS = 2, 16
NW = NC * NS

NCW = 8
XC = 400
XCPW = 31
XSPAN = XC * XCPW
XTAIL = (N_NODES - NCW * XSPAN) // XC

PC = 128
CPWC = 33
CPWP = 54
CSPAN = CPWC * PC
PSPAN = CPWP * PC
PBASE = NCW * CSPAN
PT = 80
NTAIL = (N_POOL - PBASE - (NW - NCW) * PSPAN) // PT


def _sc_body(x_hbm, i0_hbm, i1_hbm, out_hbm,
             i0v, i1v, a_v, b_v, xsh, gsem0, gsem1, ssem0, ssem1):
    w = lax.axis_index("s") * NC + lax.axis_index("c")
    gsem = [gsem0, gsem1]
    ssem = [ssem0, ssem1]
    av = [a_v.at[0], a_v.at[1]]
    bv = [b_v.at[0], b_v.at[1]]

    def run_pool(base, cpw):
        pltpu.async_copy(i0_hbm.at[pl.ds(base, cpw * PC)],
                         i0v.at[pl.ds(0, cpw * PC)], gsem[0])
        pltpu.async_copy(i1_hbm.at[pl.ds(base, cpw * PC)],
                         i1v.at[pl.ds(0, cpw * PC)], gsem[1])
        pltpu.make_async_copy(i0_hbm.at[pl.ds(base, cpw * PC)],
                              i0v.at[pl.ds(0, cpw * PC)], gsem[0]).wait()
        pltpu.make_async_copy(i1_hbm.at[pl.ds(base, cpw * PC)],
                              i1v.at[pl.ds(0, cpw * PC)], gsem[1]).wait()

        def gather_descs(k, p):
            off = k * PC
            return [
                pltpu.make_async_copy(x_hbm.at[i0v.at[pl.ds(off, PC)]],
                                      av[p], gsem[p]),
                pltpu.make_async_copy(x_hbm.at[i1v.at[pl.ds(off, PC)]],
                                      bv[p], gsem[p]),
            ]

        def fire(k, p):
            for d in gather_descs(k, p):
                d.start()

        def compute(p):
            def row(i, carry):
                for j in range(D // 16):
                    s = pl.ds(j * 16, 16)
                    a_v[p, i, s] = (a_v[p, i, s] + b_v[p, i, s]) * 0.5
                return carry

            lax.fori_loop(0, PC, row, 0)

        def consume(k, p):
            for d in gather_descs(k, p):
                d.wait()
            compute(p)
            pltpu.async_copy(av[p],
                             out_hbm.at[pl.ds(N_NODES + base + k * PC, PC), :],
                             ssem[p])

        def wait_store(k, p):
            pltpu.make_async_copy(av[p],
                                  out_hbm.at[pl.ds(N_NODES + base + k * PC, PC), :],
                                  ssem[p]).wait()

        fire(0, 0)

        def pipe(t, carry):
            k1 = 2 * t + 1

            @pl.when(t >= 1)
            def _():
                wait_store(k1 - 2, 1)

            fire(k1, 1)
            consume(k1 - 1, 0)

            k2 = 2 * t + 2
            wait_store(k2 - 2, 0)
            fire(k2, 0)
            consume(k2 - 1, 1)
            return carry

        if cpw % 2 == 0:
            lax.fori_loop(0, (cpw - 2) // 2, pipe, 0)
            wait_store(cpw - 3, 1)
            fire(cpw - 1, 1)
            consume(cpw - 2, 0)
            consume(cpw - 1, 1)
            wait_store(cpw - 2, 0)
            wait_store(cpw - 1, 1)
        else:
            lax.fori_loop(0, (cpw - 1) // 2, pipe, 0)
            consume(cpw - 1, 0)
            wait_store(cpw - 2, 1)
            wait_store(cpw - 1, 0)

    @pl.when(w < NCW)
    def _():
        base = w * XSPAN
        sidx = lax.axis_index("s")
        xb = [xsh.at[sidx, 0], xsh.at[sidx, 1]]

        def fire(k, p):
            pltpu.async_copy(x_hbm.at[pl.ds(base + k * XC, XC), :], xb[p],
                             gsem[p])

        def consume(k, p):
            pltpu.make_async_copy(x_hbm.at[pl.ds(base + k * XC, XC), :],
                                  xb[p], gsem[p]).wait()
            pltpu.async_copy(xb[p], out_hbm.at[pl.ds(base + k * XC, XC), :],
                             ssem[p])

        def wait_store(k, p):
            pltpu.make_async_copy(xb[p],
                                  out_hbm.at[pl.ds(base + k * XC, XC), :],
                                  ssem[p]).wait()

        fire(0, 0)

        def pipe(t, carry):
            k1 = 2 * t + 1

            @pl.when(t >= 1)
            def _():
                wait_store(k1 - 2, 1)

            fire(k1, 1)
            consume(k1 - 1, 0)

            k2 = 2 * t + 2
            wait_store(k2 - 2, 0)
            fire(k2, 0)
            consume(k2 - 1, 1)
            return carry

        lax.fori_loop(0, (XCPW - 1) // 2, pipe, 0)
        consume(XCPW - 1, 0)
        wait_store(XCPW - 2, 1)
        wait_store(XCPW - 1, 0)

        @pl.when(w < XTAIL)
        def _():
            tb = NCW * XSPAN + w * XC
            pltpu.sync_copy(x_hbm.at[pl.ds(tb, XC), :], xb[0])
            pltpu.sync_copy(xb[0], out_hbm.at[pl.ds(tb, XC), :])

        run_pool(w * CSPAN, CPWC)

    @pl.when(w >= NCW)
    def _():
        run_pool(PBASE + (w - NCW) * PSPAN, CPWP)

    @pl.when(w >= 4)
    def _():
        @pl.when(w - 4 < NTAIL)
        def _():
            wt = w - 4
            tbase = PBASE + (NW - NCW) * PSPAN + wt * PT
            av0 = a_v.at[0, pl.ds(0, PT), :]
            bv0 = b_v.at[0, pl.ds(0, PT), :]
            pltpu.sync_copy(i0_hbm.at[pl.ds(tbase, PT)], i0v.at[pl.ds(0, PT)])
            pltpu.sync_copy(i1_hbm.at[pl.ds(tbase, PT)], i1v.at[pl.ds(0, PT)])
            pltpu.async_copy(x_hbm.at[i0v.at[pl.ds(0, PT)]], av0, gsem[0])
            pltpu.async_copy(x_hbm.at[i1v.at[pl.ds(0, PT)]], bv0, gsem[0])
            pltpu.make_async_copy(x_hbm.at[i0v.at[pl.ds(0, PT)]], av0,
                                  gsem[0]).wait()
            pltpu.make_async_copy(x_hbm.at[i1v.at[pl.ds(0, PT)]], bv0,
                                  gsem[0]).wait()

            def trow(i, carry):
                for j in range(D // 16):
                    s = pl.ds(j * 16, 16)
                    a_v[0, i, s] = (a_v[0, i, s] + b_v[0, i, s]) * 0.5
                return carry

            lax.fori_loop(0, PT, trow, 0)
            pltpu.sync_copy(av0, out_hbm.at[pl.ds(N_NODES + tbase, PT), :])


def kernel(X, pool_idx):
    idx0 = pool_idx[:, 0]
    idx1 = pool_idx[:, 1]
    mesh = plsc.VectorSubcoreMesh(core_axis_name="c", subcore_axis_name="s")
    f = pl.kernel(
        _sc_body,
        out_type=jax.ShapeDtypeStruct((N_NODES + N_POOL, D), jnp.float32),
        mesh=mesh,
        scratch_types=[
            pltpu.VMEM((CPWP * PC,), jnp.int32),
            pltpu.VMEM((CPWP * PC,), jnp.int32),
            pltpu.VMEM((2, PC, D), jnp.float32),
            pltpu.VMEM((2, PC, D), jnp.float32),
            pltpu.VMEM_SHARED((NCW // NC, 2, XC, D), jnp.float32),
            pltpu.SemaphoreType.DMA,
            pltpu.SemaphoreType.DMA,
            pltpu.SemaphoreType.DMA,
            pltpu.SemaphoreType.DMA,
        ],
    )
    return f(X, idx0, idx1)
